# Initial kernel scaffold; baseline (speedup 1.0000x reference)
import jax
import jax.numpy as jnp
from jax.experimental import pallas as pl


def kernel(molecule_atoms, W, N):
    raise NotImplementedError("write your pallas kernel here")


# speedup vs baseline: 1.1682x; 1.1682x over previous
import functools

import jax
import jax.numpy as jnp
import numpy as np
from jax import lax
from jax.experimental import pallas as pl
from jax.experimental.pallas import tpu as pltpu
from jax.experimental.pallas import tpu_sc as plsc

B, A, F = 1024, 128, 75
NCLS = 16
EMB = 33
OTH0, OTH = 44, 31
OUT = 64
OUTP = 65
ROWS = B * A
NW = 32
RPW = ROWS // NW
CHUNK = 256
NCH = RPW // CHUNK
GPC = CHUNK // 16

_MAGIC = np.int32(0x5F3759DF)


def _rsqrt(s):
    i = plsc.bitcast(s, jnp.int32)
    i = _MAGIC - lax.shift_right_logical(i, 1)
    y = plsc.bitcast(i, jnp.float32)
    for _ in range(3):
        y = y * (1.5 - 0.5 * s * y * y)
    return y


def _splat_i32(v):
    return jnp.full((16,), v, jnp.int32)


def _sc_body(x_hbm, wt_hbm, n_hbm, out_hbm, inb, outb, wb, nb):
    wid = lax.axis_index("s") * 2 + lax.axis_index("c")
    pltpu.sync_copy(wt_hbm, wb)
    pltpu.sync_copy(n_hbm, nb)
    nvec = nb[...]
    lanes = lax.iota(jnp.int32, 16)
    row0w = wid * RPW

    def group_body(row0, g, _):
        r = g * 16 + lanes
        keep = (row0 + r) < nvec
        m = plsc.load_gather(inb, [r, _splat_i32(0)])
        p = jnp.zeros((16,), jnp.int32)
        for j in range(1, NCLS):
            v = plsc.load_gather(inb, [r, _splat_i32(j)])
            gt = v > m
            m = jnp.where(gt, v, m)
            p = jnp.where(gt, jnp.int32(j), p)
        for c in range(EMB):
            e = plsc.load_gather(wb, [_splat_i32(c), p])
            e = jnp.where(keep, e, 0.0)
            plsc.store_scatter(outb, [r, _splat_i32(c)], e)
        s = jnp.zeros((16,), jnp.float32)
        for j in range(OTH):
            v = plsc.load_gather(inb, [r, _splat_i32(OTH0 + j)])
            s = s + v * v
        y = _rsqrt(s)
        den = jnp.maximum(s * y, 1e-12)
        inv = jnp.where(keep, 1.0 / den, 0.0)
        for j in range(OTH):
            v = plsc.load_gather(inb, [r, _splat_i32(OTH0 + j)])
            plsc.store_scatter(outb, [r, _splat_i32(EMB + j)], v * inv)
        return 0

    def chunk_body(ci, _):
        row0 = row0w + ci * CHUNK
        pltpu.sync_copy(x_hbm.at[pl.ds(row0, CHUNK), :], inb)
        lax.fori_loop(0, GPC, functools.partial(group_body, row0), 0)
        pltpu.sync_copy(outb, out_hbm.at[pl.ds(row0, CHUNK), :])
        return 0

    lax.fori_loop(0, NCH, chunk_body, 0)


@jax.jit
def _sc_call(x, wt, n_arr):
    mesh = plsc.VectorSubcoreMesh(core_axis_name="c", subcore_axis_name="s",
                                  num_cores=2, num_subcores=16)
    run = pl.kernel(
        _sc_body,
        out_type=jax.ShapeDtypeStruct((ROWS, OUT), jnp.float32),
        mesh=mesh,
        scratch_types=[
            pltpu.VMEM((CHUNK, F), jnp.float32),
            pltpu.VMEM((CHUNK, OUT), jnp.float32),
            pltpu.VMEM((EMB, NCLS), jnp.float32),
            pltpu.VMEM((16,), jnp.int32),
        ],
        compiler_params=pltpu.CompilerParams(needs_layout_passes=False),
    )
    return run(x, wt, n_arr)


def kernel(molecule_atoms, W, N):
    x = molecule_atoms.reshape(ROWS, F)
    wt = W.T
    n_arr = jnp.full((16,), jnp.int32(N) * A, jnp.int32)
    out = _sc_call(x, wt, n_arr)
    return out.reshape(B, A, OUT)


# speedup vs baseline: 4.8290x; 4.1338x over previous
import functools

import jax
import jax.numpy as jnp
import numpy as np
from jax import lax
from jax.experimental import pallas as pl
from jax.experimental.pallas import tpu as pltpu
from jax.experimental.pallas import tpu_sc as plsc

B, A, F = 1024, 128, 75
NCLS = 16
EMB = 33
OTH0, OTH = 44, 31
OUT = 64
NW = 32
MPW = B // NW
MPC = 2
NCH = MPW // MPC
FPAD = 80

_MAGIC = np.int32(0x5F3759DF)

_GDN = lax.GatherDimensionNumbers(offset_dims=(), collapsed_slice_dims=(0,),
                                  start_index_map=(0,))


def _rsqrt(s):
    i = plsc.bitcast(s, jnp.int32)
    i = _MAGIC - lax.shift_right_logical(i, 1)
    y = plsc.bitcast(i, jnp.float32)
    for _ in range(3):
        y = y * (1.5 - 0.5 * s * y * y)
    return y


def _take16(col, p):
    return lax.gather(col, p[:, None], _GDN, (1,),
                      mode=lax.GatherScatterMode.PROMISE_IN_BOUNDS)


def _compute_chunk(mol0, nvec, wb, inref, outref):
    for m in range(MPC):
        keep = (mol0 + m) < nvec
        for a0 in range(0, A, 16):
            mx = inref[m * FPAD, pl.ds(a0, 16)]
            p = jnp.zeros((16,), jnp.int32)
            for j in range(1, NCLS):
                v = inref[m * FPAD + j, pl.ds(a0, 16)]
                gt = v > mx
                mx = jnp.where(gt, v, mx)
                p = jnp.where(gt, jnp.int32(j), p)
            for c in range(EMB):
                e = _take16(wb[pl.ds(c * 16, 16)], p)
                outref[m * OUT + c, pl.ds(a0, 16)] = jnp.where(keep, e, 0.0)
            s = jnp.zeros((16,), jnp.float32)
            for j in range(OTH):
                v = inref[m * FPAD + OTH0 + j, pl.ds(a0, 16)]
                s = s + v * v
            y = _rsqrt(s)
            den = jnp.maximum(s * y, 1e-12)
            inv = jnp.where(keep, 1.0 / den, 0.0)
            for j in range(OTH):
                v = inref[m * FPAD + OTH0 + j, pl.ds(a0, 16)]
                outref[m * OUT + EMB + j, pl.ds(a0, 16)] = v * inv


def _sc_body(x_hbm, wt_hbm, n_hbm, out_hbm,
             in0, in1, ou0, ou1, wb, nb, si0, si1, so0, so1):
    wid = lax.axis_index("s") * 2 + lax.axis_index("c")
    mol0w = wid * MPW
    ins, outs, sin, sout = (in0, in1), (ou0, ou1), (si0, si1), (so0, so1)
    pltpu.sync_copy(wt_hbm, wb)
    pltpu.sync_copy(n_hbm, nb)
    nvec = nb[...]

    def start_in(ci, b):
        for m in range(MPC):
            pltpu.async_copy(x_hbm.at[mol0w + ci * MPC + m, :, :],
                             ins[b].at[pl.ds(m * FPAD, F), :], sin[b])

    def wait_in(b):
        for m in range(MPC):
            pltpu.make_async_copy(x_hbm.at[0, :, :],
                                  ins[b].at[pl.ds(m * FPAD, F), :],
                                  sin[b]).wait()

    def start_out(ci, b):
        pltpu.async_copy(outs[b],
                         out_hbm.at[pl.ds((mol0w + ci * MPC) * OUT,
                                          MPC * OUT), :],
                         sout[b])

    def wait_out(b):
        pltpu.make_async_copy(outs[b],
                              out_hbm.at[pl.ds(0, MPC * OUT), :],
                              sout[b]).wait()

    start_in(0, 0)
    start_in(1, 1)

    def pair_body(k, _):
        for b in (0, 1):
            ci = 2 * k + b
            wait_in(b)

            @pl.when(k > 0)
            def _():
                wait_out(b)

            _compute_chunk(mol0w + ci * MPC, nvec, wb, ins[b], outs[b])
            start_out(ci, b)

            @pl.when(ci + 2 < NCH)
            def _():
                start_in(ci + 2, b)
        return 0

    lax.fori_loop(0, NCH // 2, pair_body, 0)
    wait_out(0)
    wait_out(1)


@jax.jit
def _sc_call(x_t, wt, n_arr):
    mesh = plsc.VectorSubcoreMesh(core_axis_name="c", subcore_axis_name="s",
                                  num_cores=2, num_subcores=16)
    run = pl.kernel(
        _sc_body,
        out_type=jax.ShapeDtypeStruct((B * OUT, A), jnp.float32),
        mesh=mesh,
        scratch_types=[
            pltpu.VMEM((MPC * FPAD, A), jnp.float32),
            pltpu.VMEM((MPC * FPAD, A), jnp.float32),
            pltpu.VMEM((MPC * OUT, A), jnp.float32),
            pltpu.VMEM((MPC * OUT, A), jnp.float32),
            pltpu.VMEM((EMB * NCLS,), jnp.float32),
            pltpu.VMEM((16,), jnp.int32),
            pltpu.SemaphoreType.DMA,
            pltpu.SemaphoreType.DMA,
            pltpu.SemaphoreType.DMA,
            pltpu.SemaphoreType.DMA,
        ],
        compiler_params=pltpu.CompilerParams(needs_layout_passes=False),
    )
    return run(x_t, wt, n_arr)


def kernel(molecule_atoms, W, N):
    x_t = jnp.transpose(molecule_atoms, (0, 2, 1))
    wt = W.T.reshape(EMB * NCLS)
    n_arr = jnp.full((16,), jnp.int32(N), jnp.int32)
    out2d = _sc_call(x_t, wt, n_arr)
    return jnp.transpose(out2d.reshape(B, OUT, A), (0, 2, 1))
